# baseline (speedup 1.0000x reference)
import jax
import jax.numpy as jnp
from jax.experimental import pallas as pl


def _pos_embed_tile_kernel(x_ref, col_ref, row_ref, out_ref):
    num_clips, d = col_ref.shape
    hw = x_ref.shape[1]
    w = hw // num_clips

    k_iota = jax.lax.broadcasted_iota(jnp.int32, (num_clips, hw), 0)
    p_iota = jax.lax.broadcasted_iota(jnp.int32, (num_clips, hw), 1)

    onehot_col = (x_ref[:] == k_iota).astype(jnp.float32)
    onehot_row = ((p_iota // w) == k_iota).astype(jnp.float32)

    dn = (((0,), (0,)), ((), ()))
    col_part = jax.lax.dot_general(col_ref[:], onehot_col, dn,
                                   preferred_element_type=jnp.float32,
                                   precision=jax.lax.Precision.HIGHEST)
    row_part = jax.lax.dot_general(row_ref[:], onehot_row, dn,
                                   preferred_element_type=jnp.float32,
                                   precision=jax.lax.Precision.HIGHEST)

    out_ref[0, :d, :] = col_part
    out_ref[0, d:, :] = row_part


def kernel(x, col_embed, row_embed):
    h, w = x.shape
    num_clips, d = col_embed.shape
    b = h
    hw = h * w

    x_flat = x.reshape(1, hw)

    out_flat = pl.pallas_call(
        _pos_embed_tile_kernel,
        grid=(b,),
        in_specs=[
            pl.BlockSpec((1, hw), lambda i: (0, 0)),
            pl.BlockSpec((num_clips, d), lambda i: (0, 0)),
            pl.BlockSpec((num_clips, d), lambda i: (0, 0)),
        ],
        out_specs=pl.BlockSpec((1, 2 * d, hw), lambda i: (i, 0, 0)),
        out_shape=jax.ShapeDtypeStruct((b, 2 * d, hw), jnp.float32),
    )(x_flat, col_embed, row_embed)

    return out_flat.reshape(b, 2 * d, h, w)


# speedup vs baseline: 1.4220x; 1.4220x over previous
import jax
import jax.numpy as jnp
from jax.experimental import pallas as pl
from jax.experimental.pallas import tpu as pltpu


def _pos_embed_kernel(x_ref, col_ref, row_ref, out_ref, tile, sem):
    num_clips, d = col_ref.shape
    hw = x_ref.shape[1]
    w = hw // num_clips
    b = out_ref.shape[0]

    k_iota = jax.lax.broadcasted_iota(jnp.int32, (num_clips, hw), 0)
    p_iota = jax.lax.broadcasted_iota(jnp.int32, (num_clips, hw), 1)

    onehot_col = (x_ref[:] == k_iota).astype(jnp.float32)
    onehot_row = ((p_iota // w) == k_iota).astype(jnp.float32)

    dn = (((0,), (0,)), ((), ()))
    tile[:d, :] = jax.lax.dot_general(col_ref[:], onehot_col, dn,
                                      preferred_element_type=jnp.float32,
                                      precision=jax.lax.Precision.HIGHEST)
    tile[d:, :] = jax.lax.dot_general(row_ref[:], onehot_row, dn,
                                      preferred_element_type=jnp.float32,
                                      precision=jax.lax.Precision.HIGHEST)

    copies = [pltpu.make_async_copy(tile, out_ref.at[i], sem) for i in range(b)]
    for c in copies:
        c.start()
    for c in copies:
        c.wait()


def kernel(x, col_embed, row_embed):
    h, w = x.shape
    num_clips, d = col_embed.shape
    b = h
    hw = h * w

    x_flat = x.reshape(1, hw)

    out_flat = pl.pallas_call(
        _pos_embed_kernel,
        in_specs=[
            pl.BlockSpec(memory_space=pltpu.MemorySpace.VMEM),
            pl.BlockSpec(memory_space=pltpu.MemorySpace.VMEM),
            pl.BlockSpec(memory_space=pltpu.MemorySpace.VMEM),
        ],
        out_specs=pl.BlockSpec(memory_space=pltpu.MemorySpace.HBM),
        out_shape=jax.ShapeDtypeStruct((b, 2 * d, hw), jnp.float32),
        scratch_shapes=[
            pltpu.VMEM((2 * d, hw), jnp.float32),
            pltpu.SemaphoreType.DMA,
        ],
    )(x_flat, col_embed, row_embed)

    return out_flat.reshape(b, 2 * d, h, w)
